# Initial kernel scaffold; baseline (speedup 1.0000x reference)
import jax
import jax.numpy as jnp
from jax.experimental import pallas as pl


def kernel(points, features, W0, b0, gamma0, beta0, W1, b1, gamma1, beta1, W2, b2, gamma2, beta2):
    raise NotImplementedError("write your pallas kernel here")


# speedup vs baseline: 13.1750x; 13.1750x over previous
import functools

import jax
import jax.numpy as jnp
from jax import lax
from jax.experimental import pallas as pl
from jax.experimental.pallas import tpu as pltpu
from jax.experimental.pallas import tpu_sc as plsc

B = 4
N = 8192
M = 1024
S = 16
R2 = 1.0
BM = B * M
P = BM * S


def _fps_body(pts_ref, idx_ref, np_ref):
    pts = pts_ref[...]
    iota = lax.broadcasted_iota(jnp.int32, (B, N), 1)

    def coords_of(last):
        onehot = iota == last[:, None]
        return jnp.sum(jnp.where(onehot[:, None, :], pts, 0.0), axis=2)

    def dist_to(c):
        d0 = pts[:, 0, :] - c[:, 0:1]
        d1 = pts[:, 1, :] - c[:, 1:2]
        d2 = pts[:, 2, :] - c[:, 2:3]
        return (d0 * d0 + d1 * d1) + d2 * d2

    idx_ref[pl.ds(0, 1), :] = jnp.zeros((1, B), jnp.int32)

    def body(i, carry):
        dists, last = carry
        c = coords_of(last)
        np_ref[pl.ds(i - 1, 1)] = c[None]
        d = dist_to(c)
        dists = jnp.minimum(dists, d)
        mx = jnp.max(dists, axis=1, keepdims=True)
        nxt = jnp.min(jnp.where(dists == mx, iota, N), axis=1).astype(jnp.int32)
        idx_ref[pl.ds(i, 1), :] = nxt[None, :]
        return dists, nxt

    dists0 = jnp.full((B, N), 1e10, jnp.float32)
    last0 = jnp.zeros((B,), jnp.int32)
    _, last = lax.fori_loop(1, M, body, (dists0, last0))
    np_ref[pl.ds(M - 1, 1)] = coords_of(last)[None]


def _run_fps(points):
    idx_mb, np_mb = pl.pallas_call(
        _fps_body,
        out_shape=(
            jax.ShapeDtypeStruct((M, B), jnp.int32),
            jax.ShapeDtypeStruct((M, B, 3), jnp.float32),
        ),
    )(points)
    return idx_mb, np_mb


_MBLK = 256


def _ballq_body(pts_ref, np_ref, ind_ref):
    pts = pts_ref[0]
    npb = np_ref[0]
    xx = jnp.sum(npb * npb, axis=1, keepdims=True)
    yy = jnp.sum(pts * pts, axis=0, keepdims=True)
    cross = jnp.dot(npb, pts, preferred_element_type=jnp.float32)
    d2 = jnp.maximum(xx + yy - 2.0 * cross, 0.0)
    iota = lax.broadcasted_iota(jnp.int32, (_MBLK, N), 1)
    cand = jnp.where(d2 < R2, iota, N)
    sels = []
    for _ in range(S):
        m = jnp.min(cand, axis=1)
        sels.append(m[:, None])
        cand = jnp.where(cand == m[:, None], N, cand)
    sel = jnp.concatenate(sels, axis=1)
    first = sel[:, 0:1]
    first = jnp.where(first >= N, 0, first)
    sel = jnp.where(sel >= N, first, sel)
    ind_ref[0] = sel


def _run_ballq(points, new_points_bm3):
    return pl.pallas_call(
        _ballq_body,
        grid=(B, M // _MBLK),
        in_specs=[
            pl.BlockSpec((1, 3, N), lambda b, m: (b, 0, 0)),
            pl.BlockSpec((1, _MBLK, 3), lambda b, m: (b, m, 0)),
        ],
        out_specs=pl.BlockSpec((1, _MBLK, S), lambda b, m: (b, m, 0)),
        out_shape=jax.ShapeDtypeStruct((B, M, S), jnp.int32),
    )(points, new_points_bm3)


_NW = 32
_RPW = P // _NW
_CH = 128
_NCH = _RPW // _CH


def _sc_gather_body(table_hbm, ind_hbm, out_hbm, idx_v, rows_v, sem):
    wid = lax.axis_index("s") * 2 + lax.axis_index("c")
    base = wid * _NCH
    pltpu.sync_copy(ind_hbm.at[pl.ds(base, _NCH)], idx_v)
    boff = (wid // (_NW // B)) * N
    for j in range(_NCH):
        for k in range(_CH // 16):
            sl = pl.ds(k * 16, 16)
            idx_v[j, sl] = idx_v[j, sl] + boff
    copies = [
        pltpu.async_copy(table_hbm.at[idx_v.at[j]], rows_v.at[j], sem)
        for j in range(_NCH)
    ]
    for c in copies:
        c.wait()
    pltpu.sync_copy(rows_v, out_hbm.at[pl.ds(base, _NCH)])


def _run_sc_gather(table, ind_flat2d):
    mesh = plsc.VectorSubcoreMesh(core_axis_name="c", subcore_axis_name="s")
    f = functools.partial(
        pl.kernel,
        mesh=mesh,
        compiler_params=pltpu.CompilerParams(use_tc_tiling_on_sc=False),
        out_type=jax.ShapeDtypeStruct((P // _CH, _CH, 8), jnp.float32),
        scratch_types=[
            pltpu.VMEM((_NCH, _CH), jnp.int32),
            pltpu.VMEM((_NCH, _CH, 8), jnp.float32),
            pltpu.SemaphoreType.DMA,
        ],
    )(_sc_gather_body)
    return f(table, ind_flat2d)


def _mlp_body(g_ref, c_ref, w0_ref, w1_ref, w2_ref, p0_ref, p1_ref, p2_ref,
              out_ref):
    x = g_ref[...] - c_ref[...]

    def layer(x, w_ref, p_ref, cout):
        y = jnp.dot(x, w_ref[...], preferred_element_type=jnp.float32)
        y = y + p_ref[0, 0:S * cout][None, :]
        tot = jnp.zeros((1, cout), jnp.float32)
        sq = jnp.zeros((1, cout), jnp.float32)
        for s in range(S):
            blk = y[:, s * cout:(s + 1) * cout]
            tot = tot + jnp.sum(blk, axis=0, keepdims=True)
            sq = sq + jnp.sum(blk * blk, axis=0, keepdims=True)
        cnt = jnp.float32(P)
        mean = tot / cnt
        var = sq / cnt - mean * mean
        scale = p_ref[1][None, 0:cout] / jnp.sqrt(var + 1e-3)
        shift = p_ref[2][None, 0:cout] - mean * scale
        outs = []
        for s in range(S):
            blk = y[:, s * cout:(s + 1) * cout]
            outs.append(jnp.maximum(blk * scale + shift, 0.0))
        return jnp.concatenate(outs, axis=1)

    x = layer(x, w0_ref, p0_ref, 32)
    x = layer(x, w1_ref, p1_ref, 32)
    x = layer(x, w2_ref, p2_ref, 64)
    red = x[:, 0:64]
    for s in range(1, S):
        red = jnp.maximum(red, x[:, s * 64:(s + 1) * 64])
    out_ref[...] = red


def _run_mlp(g128, c128, w0bd, w1bd, w2bd, p0, p1, p2):
    return pl.pallas_call(
        _mlp_body,
        out_shape=jax.ShapeDtypeStruct((BM, 64), jnp.float32),
    )(g128, c128, w0bd, w1bd, w2bd, p0, p1, p2)


def kernel(points, features, W0, b0, gamma0, beta0, W1, b1, gamma1, beta1,
           W2, b2, gamma2, beta2):
    idx_mb, np_mb3 = _run_fps(points)
    new_points = jnp.transpose(np_mb3, (1, 2, 0))
    np_bm3 = jnp.transpose(np_mb3, (1, 0, 2))

    ind = _run_ballq(points, np_bm3)

    table = jnp.concatenate(
        [jnp.transpose(points, (0, 2, 1)),
         jnp.transpose(features, (0, 2, 1)),
         jnp.zeros((B, N, 2), jnp.float32)], axis=2).reshape(B * N, 8)
    gathered = _run_sc_gather(table, ind.reshape(P // _CH, _CH))
    g128 = gathered.reshape(BM, S * 8)

    c8 = jnp.concatenate(
        [np_bm3.reshape(BM, 3), jnp.zeros((BM, 5), jnp.float32)], axis=1)
    c128 = jnp.tile(c8, (1, S))

    eye = jnp.eye(S, dtype=jnp.float32)
    w0p = jnp.concatenate([W0, jnp.zeros((2, 32), jnp.float32)], axis=0)
    w0bd = jnp.kron(eye, w0p)
    w1bd = jnp.kron(eye, W1)
    w2bd = jnp.kron(eye, W2)

    def params(bias, gamma, beta, cout):
        biasr = jnp.tile(bias, S)
        biasr = jnp.pad(biasr, (0, S * 64 - S * cout))
        g = jnp.pad(gamma, (0, 64 - cout))
        be = jnp.pad(beta, (0, 64 - cout))
        g = jnp.pad(g[None, :], ((0, 0), (0, S * 64 - 64)))[0]
        be = jnp.pad(be[None, :], ((0, 0), (0, S * 64 - 64)))[0]
        return jnp.stack([biasr, g, be], axis=0)

    p0 = params(b0, gamma0, beta0, 32)
    p1 = params(b1, gamma1, beta1, 32)
    p2 = params(b2, gamma2, beta2, 64)

    pooled = _run_mlp(g128, c128, w0bd, w1bd, w2bd, p0, p1, p2)
    new_features = jnp.transpose(pooled.reshape(B, M, 64), (0, 2, 1))
    return new_points, new_features


# speedup vs baseline: 25.3126x; 1.9213x over previous
import functools

import jax
import jax.numpy as jnp
from jax import lax
from jax.experimental import pallas as pl
from jax.experimental.pallas import tpu as pltpu
from jax.experimental.pallas import tpu_sc as plsc

B = 4
N = 8192
M = 1024
S = 16
R2 = 1.0
BM = B * M
P = BM * S


_NR = 8
_NC = N // _NR


def _fps_body(pts_ref, idx_ref, np_ref):
    pts = pts_ref[...]
    iota = (lax.broadcasted_iota(jnp.int32, (_NR, _NC), 0) * _NC
            + lax.broadcasted_iota(jnp.int32, (_NR, _NC), 1))

    def coords_of(last):
        onehot = iota[None] == last[:, None, None]
        return jnp.sum(jnp.where(onehot[:, None], pts, 0.0), axis=(2, 3))

    def dist_to(c):
        d0 = pts[:, 0] - c[:, 0, None, None]
        d1 = pts[:, 1] - c[:, 1, None, None]
        d2 = pts[:, 2] - c[:, 2, None, None]
        return (d0 * d0 + d1 * d1) + d2 * d2

    idx_ref[pl.ds(0, 1), :] = jnp.zeros((1, B), jnp.int32)

    def body(i, carry):
        dists, last = carry
        c = coords_of(last)
        np_ref[pl.ds(i - 1, 1)] = c[None]
        dists = jnp.minimum(dists, dist_to(c))
        mx = jnp.max(dists, axis=(1, 2))
        eq = dists == mx[:, None, None]
        nxt = jnp.min(jnp.where(eq, iota[None], N), axis=(1, 2)).astype(jnp.int32)
        idx_ref[pl.ds(i, 1), :] = nxt[None, :]
        return dists, nxt

    dists0 = jnp.full((B, _NR, _NC), 1e10, jnp.float32)
    last0 = jnp.zeros((B,), jnp.int32)
    _, last = lax.fori_loop(1, M, body, (dists0, last0))
    np_ref[pl.ds(M - 1, 1)] = coords_of(last)[None]


def _run_fps(points):
    idx_mb, np_mb = pl.pallas_call(
        _fps_body,
        out_shape=(
            jax.ShapeDtypeStruct((M, B), jnp.int32),
            jax.ShapeDtypeStruct((M, B, 3), jnp.float32),
        ),
    )(points.reshape(B, 3, _NR, _NC))
    return idx_mb, np_mb


_MBLK = 256


def _ballq_body(pts_ref, np_ref, ind_ref):
    pts = pts_ref[0]
    npb = np_ref[0]
    xx = jnp.sum(npb * npb, axis=1, keepdims=True)
    yy = jnp.sum(pts * pts, axis=0, keepdims=True)
    cross = jnp.dot(npb, pts, preferred_element_type=jnp.float32)
    d2 = jnp.maximum(xx + yy - 2.0 * cross, 0.0)
    iota = lax.broadcasted_iota(jnp.int32, (_MBLK, N), 1)
    cand = jnp.where(d2 < R2, iota, N)
    sels = []
    for _ in range(S):
        m = jnp.min(cand, axis=1)
        sels.append(m[:, None])
        cand = jnp.where(cand == m[:, None], N, cand)
    sel = jnp.concatenate(sels, axis=1)
    first = sel[:, 0:1]
    first = jnp.where(first >= N, 0, first)
    sel = jnp.where(sel >= N, first, sel)
    ind_ref[0] = sel


def _run_ballq(points, new_points_bm3):
    return pl.pallas_call(
        _ballq_body,
        grid=(B, M // _MBLK),
        in_specs=[
            pl.BlockSpec((1, 3, N), lambda b, m: (b, 0, 0)),
            pl.BlockSpec((1, _MBLK, 3), lambda b, m: (b, m, 0)),
        ],
        out_specs=pl.BlockSpec((1, _MBLK, S), lambda b, m: (b, m, 0)),
        out_shape=jax.ShapeDtypeStruct((B, M, S), jnp.int32),
    )(points, new_points_bm3)


_NW = 32
_RPW = P // _NW
_CH = 128
_NCH = _RPW // _CH


def _sc_gather_body(table_hbm, ind_hbm, out_hbm, idx_v, rows_v, sem):
    wid = lax.axis_index("s") * 2 + lax.axis_index("c")
    base = wid * _NCH
    pltpu.sync_copy(ind_hbm.at[pl.ds(base, _NCH)], idx_v)
    boff = (wid // (_NW // B)) * N
    for j in range(_NCH):
        for k in range(_CH // 16):
            sl = pl.ds(k * 16, 16)
            idx_v[j, sl] = idx_v[j, sl] + boff
    copies = [
        pltpu.async_copy(table_hbm.at[idx_v.at[j]], rows_v.at[j], sem)
        for j in range(_NCH)
    ]
    for c in copies:
        c.wait()
    pltpu.sync_copy(rows_v, out_hbm.at[pl.ds(base, _NCH)])


def _run_sc_gather(table, ind_flat2d):
    mesh = plsc.VectorSubcoreMesh(core_axis_name="c", subcore_axis_name="s")
    f = functools.partial(
        pl.kernel,
        mesh=mesh,
        compiler_params=pltpu.CompilerParams(use_tc_tiling_on_sc=False),
        out_type=jax.ShapeDtypeStruct((P // _CH, _CH, 8), jnp.float32),
        scratch_types=[
            pltpu.VMEM((_NCH, _CH), jnp.int32),
            pltpu.VMEM((_NCH, _CH, 8), jnp.float32),
            pltpu.SemaphoreType.DMA,
        ],
    )(_sc_gather_body)
    return f(table, ind_flat2d)


def _mlp_body(g_ref, c_ref, w0_ref, w1_ref, w2_ref, p0_ref, p1_ref, p2_ref,
              out_ref):
    x = g_ref[...] - c_ref[...]

    def layer(x, w_ref, p_ref, cout):
        y = jnp.dot(x, w_ref[...], preferred_element_type=jnp.float32)
        y = y + p_ref[0, 0:S * cout][None, :]
        tot = jnp.zeros((1, cout), jnp.float32)
        sq = jnp.zeros((1, cout), jnp.float32)
        for s in range(S):
            blk = y[:, s * cout:(s + 1) * cout]
            tot = tot + jnp.sum(blk, axis=0, keepdims=True)
            sq = sq + jnp.sum(blk * blk, axis=0, keepdims=True)
        cnt = jnp.float32(P)
        mean = tot / cnt
        var = sq / cnt - mean * mean
        scale = p_ref[1][None, 0:cout] / jnp.sqrt(var + 1e-3)
        shift = p_ref[2][None, 0:cout] - mean * scale
        outs = []
        for s in range(S):
            blk = y[:, s * cout:(s + 1) * cout]
            outs.append(jnp.maximum(blk * scale + shift, 0.0))
        return jnp.concatenate(outs, axis=1)

    x = layer(x, w0_ref, p0_ref, 32)
    x = layer(x, w1_ref, p1_ref, 32)
    x = layer(x, w2_ref, p2_ref, 64)
    red = x[:, 0:64]
    for s in range(1, S):
        red = jnp.maximum(red, x[:, s * 64:(s + 1) * 64])
    out_ref[...] = red


def _run_mlp(g128, c128, w0bd, w1bd, w2bd, p0, p1, p2):
    return pl.pallas_call(
        _mlp_body,
        out_shape=jax.ShapeDtypeStruct((BM, 64), jnp.float32),
    )(g128, c128, w0bd, w1bd, w2bd, p0, p1, p2)


def kernel(points, features, W0, b0, gamma0, beta0, W1, b1, gamma1, beta1,
           W2, b2, gamma2, beta2):
    idx_mb, np_mb3 = _run_fps(points)
    new_points = jnp.transpose(np_mb3, (1, 2, 0))
    np_bm3 = jnp.transpose(np_mb3, (1, 0, 2))

    ind = _run_ballq(points, np_bm3)

    table = jnp.concatenate(
        [jnp.transpose(points, (0, 2, 1)),
         jnp.transpose(features, (0, 2, 1)),
         jnp.zeros((B, N, 2), jnp.float32)], axis=2).reshape(B * N, 8)
    gathered = _run_sc_gather(table, ind.reshape(P // _CH, _CH))
    g128 = gathered.reshape(BM, S * 8)

    c8 = jnp.concatenate(
        [np_bm3.reshape(BM, 3), jnp.zeros((BM, 5), jnp.float32)], axis=1)
    c128 = jnp.tile(c8, (1, S))

    eye = jnp.eye(S, dtype=jnp.float32)
    w0p = jnp.concatenate([W0, jnp.zeros((2, 32), jnp.float32)], axis=0)
    w0bd = jnp.kron(eye, w0p)
    w1bd = jnp.kron(eye, W1)
    w2bd = jnp.kron(eye, W2)

    def params(bias, gamma, beta, cout):
        biasr = jnp.tile(bias, S)
        biasr = jnp.pad(biasr, (0, S * 64 - S * cout))
        g = jnp.pad(gamma, (0, 64 - cout))
        be = jnp.pad(beta, (0, 64 - cout))
        g = jnp.pad(g[None, :], ((0, 0), (0, S * 64 - 64)))[0]
        be = jnp.pad(be[None, :], ((0, 0), (0, S * 64 - 64)))[0]
        return jnp.stack([biasr, g, be], axis=0)

    p0 = params(b0, gamma0, beta0, 32)
    p1 = params(b1, gamma1, beta1, 32)
    p2 = params(b2, gamma2, beta2, 64)

    pooled = _run_mlp(g128, c128, w0bd, w1bd, w2bd, p0, p1, p2)
    new_features = jnp.transpose(pooled.reshape(B, M, 64), (0, 2, 1))
    return new_points, new_features
